# baseline (speedup 1.0000x reference)
import functools

import jax
import jax.numpy as jnp
from jax import lax
from jax.experimental import pallas as pl
from jax.experimental.pallas import tpu as pltpu
from jax.experimental.pallas import tpu_sc as plsc

_NUM_FIELDS = 26
_VOCAB = 100000
_EMBED_DIM = 32
_BATCH = 16384

_NC = 2
_NS = 16
_NBUF = 2


def _make_body(nfields):
    def _gather_body(x_hbm, tab_hbm, out_hbm, idx_v, row_v, sem_i, sem_g, sem_w):
        e = lax.axis_index("s") * _NC + lax.axis_index("c")

        pltpu.async_copy(x_hbm.at[0], idx_v.at[0], sem_i.at[0])

        @pl.loop(0, nfields)
        def _field(f):
            b = lax.rem(f, _NBUF)
            nb = lax.rem(f + 1, _NBUF)

            @pl.when(f + 1 < nfields)
            def _prefetch_idx():
                pltpu.async_copy(x_hbm.at[f + 1], idx_v.at[nb], sem_i.at[nb])

            pltpu.make_async_copy(x_hbm.at[0], idx_v.at[b], sem_i.at[b]).wait()

            @pl.when(f >= _NBUF)
            def _reclaim():
                pltpu.make_async_copy(
                    row_v.at[b], out_hbm.at[0, 0], sem_w.at[b]
                ).wait()

            pltpu.async_copy(
                tab_hbm.at[f, e].at[idx_v.at[b]], row_v.at[b], sem_g.at[b]
            )
            pltpu.make_async_copy(
                tab_hbm.at[0, 0], row_v.at[b], sem_g.at[b]
            ).wait()
            pltpu.async_copy(row_v.at[b], out_hbm.at[f, e], sem_w.at[b])

        for b in range(min(_NBUF, nfields)):
            pltpu.make_async_copy(
                row_v.at[b], out_hbm.at[0, 0], sem_w.at[b]
            ).wait()

    return _gather_body


def _make_gather(nfields):
    return functools.partial(
        pl.kernel,
        out_type=jax.ShapeDtypeStruct(
            (nfields, _EMBED_DIM, _BATCH), jnp.float32
        ),
        mesh=plsc.VectorSubcoreMesh(
            core_axis_name="c", subcore_axis_name="s",
            num_cores=_NC, num_subcores=_NS,
        ),
        scratch_types=[
            pltpu.VMEM((_NBUF, _BATCH), jnp.int32),
            pltpu.VMEM((_NBUF, _BATCH), jnp.float32),
            pltpu.SemaphoreType.DMA((_NBUF,)),
            pltpu.SemaphoreType.DMA((_NBUF,)),
            pltpu.SemaphoreType.DMA((_NBUF,)),
        ],
        compiler_params=pltpu.CompilerParams(use_tc_tiling_on_sc=False),
    )(_make_body(nfields))


_GROUPS = (13, 13)
_GATHERS = {n: _make_gather(n) for n in set(_GROUPS)}


def kernel(x, tables):
    x_t = x.T.astype(jnp.int32)
    tab_t = jnp.transpose(tables, (0, 2, 1))
    outs = []
    f0 = 0
    for n in _GROUPS:
        outs.append(_GATHERS[n](x_t[f0:f0 + n], tab_t[f0:f0 + n]))
        f0 += n
    out_t = jnp.concatenate(outs, axis=0)
    return jnp.transpose(out_t, (2, 0, 1))


# speedup vs baseline: 1.0702x; 1.0702x over previous
import functools

import jax
import jax.numpy as jnp
from jax import lax
from jax.experimental import pallas as pl
from jax.experimental.pallas import tpu as pltpu
from jax.experimental.pallas import tpu_sc as plsc

_NUM_FIELDS = 26
_VOCAB = 100000
_EMBED_DIM = 32
_BATCH = 16384

_NC = 2
_NS = 16
_NBUF = 2


def _make_body(nfields):
    def _gather_body(x_hbm, tab_hbm, out_hbm, idx_v, row_v, sem_i, sem_g, sem_w):
        e = lax.axis_index("s") * _NC + lax.axis_index("c")

        pltpu.async_copy(x_hbm.at[0], idx_v.at[0], sem_i.at[0])

        @pl.loop(0, nfields)
        def _field(f):
            b = lax.rem(f, _NBUF)
            nb = lax.rem(f + 1, _NBUF)

            @pl.when(f + 1 < nfields)
            def _prefetch_idx():
                pltpu.async_copy(x_hbm.at[f + 1], idx_v.at[nb], sem_i.at[nb])

            pltpu.make_async_copy(x_hbm.at[0], idx_v.at[b], sem_i.at[b]).wait()

            @pl.when(f >= _NBUF)
            def _reclaim():
                pltpu.make_async_copy(
                    row_v.at[b], out_hbm.at[0, 0], sem_w.at[b]
                ).wait()

            pltpu.async_copy(
                tab_hbm.at[f, e].at[idx_v.at[b]], row_v.at[b], sem_g.at[b]
            )
            pltpu.make_async_copy(
                tab_hbm.at[0, 0], row_v.at[b], sem_g.at[b]
            ).wait()
            pltpu.async_copy(row_v.at[b], out_hbm.at[f, e], sem_w.at[b])

        for b in range(min(_NBUF, nfields)):
            pltpu.make_async_copy(
                row_v.at[b], out_hbm.at[0, 0], sem_w.at[b]
            ).wait()

    return _gather_body


def _make_gather(nfields):
    return functools.partial(
        pl.kernel,
        out_type=jax.ShapeDtypeStruct(
            (nfields, _EMBED_DIM, _BATCH), jnp.float32
        ),
        mesh=plsc.VectorSubcoreMesh(
            core_axis_name="c", subcore_axis_name="s",
            num_cores=_NC, num_subcores=_NS,
        ),
        scratch_types=[
            pltpu.VMEM((_NBUF, _BATCH), jnp.int32),
            pltpu.VMEM((_NBUF, _BATCH), jnp.float32),
            pltpu.SemaphoreType.DMA((_NBUF,)),
            pltpu.SemaphoreType.DMA((_NBUF,)),
            pltpu.SemaphoreType.DMA((_NBUF,)),
        ],
        compiler_params=pltpu.CompilerParams(use_tc_tiling_on_sc=False),
    )(_make_body(nfields))


_gather = _make_gather(_NUM_FIELDS)


def kernel(x, tables):
    x_t = x.T.astype(jnp.int32)
    tab_t = jnp.transpose(tables, (0, 2, 1))
    out_t = _gather(x_t, tab_t)
    return jnp.transpose(out_t, (2, 0, 1))
